# baseline (speedup 1.0000x reference)
import functools

import jax
import jax.numpy as jnp
from jax import lax
from jax.experimental import pallas as pl
from jax.experimental.pallas import tpu as pltpu
from jax.experimental.pallas import tpu_sc as plsc

EXTENT = 0.3
PTC = 16
CHUNK = 128
NC, NS = 2, 16
NW = NC * NS
MB = 400


def _sc_gather(featb, coords, idx):
    B = idx.shape[0]
    per_w = B // NW
    n_chunks = per_w // CHUNK
    mesh = plsc.VectorSubcoreMesh(core_axis_name="c", subcore_axis_name="s")

    @functools.partial(
        pl.kernel,
        mesh=mesh,
        out_type=(
            jax.ShapeDtypeStruct((B, featb.shape[1]), jnp.bfloat16),
            jax.ShapeDtypeStruct((B, PTC), jnp.float32),
        ),
        compiler_params=pltpu.CompilerParams(use_tc_tiling_on_sc=False),
        scratch_types=[
            pltpu.VMEM((CHUNK,), jnp.int32),
            pltpu.VMEM((CHUNK, featb.shape[1]), jnp.bfloat16),
            pltpu.VMEM((CHUNK, PTC), jnp.float32),
            pltpu.SemaphoreType.DMA,
            pltpu.SemaphoreType.DMA,
        ],
    )
    def gather_kernel(featb_hbm, coords_hbm, idx_hbm, outf_hbm, outp_hbm,
                      idx_v, rowsf_v, rowsp_v, sem_f, sem_p):
        wid = lax.axis_index("s") * NC + lax.axis_index("c")
        base = wid * per_w

        @pl.loop(0, n_chunks)
        def _(c):
            off = base + c * CHUNK
            pltpu.sync_copy(idx_hbm.at[pl.ds(off, CHUNK)], idx_v)
            cf = pltpu.async_copy(featb_hbm.at[idx_v], rowsf_v, sem_f)
            cp = pltpu.async_copy(coords_hbm.at[idx_v], rowsp_v, sem_p)
            cf.wait()
            cp.wait()
            pltpu.sync_copy(rowsf_v, outf_hbm.at[pl.ds(off, CHUNK)])
            pltpu.sync_copy(rowsp_v, outp_hbm.at[pl.ds(off, CHUNK)])

    return gather_kernel(featb, coords, idx)


def _make_tc_body(mb, d, k):
    def tc_body(gf_ref, gp_ref, outp_ref, kpt_ref, kv_ref, out_ref):
        featsb = gf_ref[...]
        pts = gp_ref[:, 0:3]
        op = outp_ref[...]
        opr = jnp.broadcast_to(op[:, None, :], (mb, d, 3)).reshape(mb * d, 3)
        sq = jnp.zeros((mb * d, 16), jnp.float32)
        for c in range(3):
            dc = pts[:, c:c + 1] - opr[:, c:c + 1]
            sq = sq + (dc - kpt_ref[c:c + 1, :]) ** 2
        w = jnp.maximum(1.0 - jnp.sqrt(sq) / EXTENT, 0.0)
        wb = w.astype(jnp.bfloat16)
        acc = jnp.zeros((mb, 128), jnp.float32)
        for j in range(k):
            p = wb[:, j:j + 1] * featsb
            wfj = p.reshape(mb, d, 128).sum(axis=1)
            acc = acc + jnp.dot(wfj, kv_ref[j],
                                preferred_element_type=jnp.float32)
        out_ref[...] = acc
    return tc_body


def kernel(points, features, output_points, neighbor_indices, k_points, k_values):
    n, f = features.shape
    m, d = neighbor_indices.shape
    k = k_values.shape[0]
    c_out = k_values.shape[2]

    featb = features.astype(jnp.bfloat16)
    coords = jnp.concatenate(
        [points, jnp.zeros((n, PTC - 3), jnp.float32)], axis=1)
    b = m * d
    grain = NW * CHUNK
    b_pad = ((b + grain - 1) // grain) * grain
    idx = jnp.pad(neighbor_indices.reshape(-1).astype(jnp.int32),
                  (0, b_pad - b))

    kpt = jnp.full((4, 16), 1e6, jnp.float32)
    kpt = kpt.at[0:3, 0:k].set(k_points.T)

    gf, gp = _sc_gather(featb, coords, idx)

    out = pl.pallas_call(
        _make_tc_body(MB, d, k),
        grid=(m // MB,),
        in_specs=[
            pl.BlockSpec((MB * d, f), lambda i: (i, 0)),
            pl.BlockSpec((MB * d, PTC), lambda i: (i, 0)),
            pl.BlockSpec((MB, 3), lambda i: (i, 0)),
            pl.BlockSpec((4, 16), lambda i: (0, 0)),
            pl.BlockSpec((k, f, c_out), lambda i: (0, 0, 0)),
        ],
        out_specs=pl.BlockSpec((MB, c_out), lambda i: (i, 0)),
        out_shape=jax.ShapeDtypeStruct((m, c_out), jnp.float32),
    )(gf, gp, output_points, kpt, k_values.astype(jnp.bfloat16))
    return out


# speedup vs baseline: 1.0031x; 1.0031x over previous
import functools

import jax
import jax.numpy as jnp
from jax import lax
from jax.experimental import pallas as pl
from jax.experimental.pallas import tpu as pltpu
from jax.experimental.pallas import tpu_sc as plsc

EXTENT = 0.3
PTC = 16
CHUNK = 128
NC, NS = 2, 16
NW = NC * NS
MB = 400


def _sc_gather(featb, coords, idx):
    B = idx.shape[0]
    per_w = B // NW
    n_chunks = per_w // CHUNK
    mesh = plsc.VectorSubcoreMesh(core_axis_name="c", subcore_axis_name="s")

    @functools.partial(
        pl.kernel,
        mesh=mesh,
        out_type=(
            jax.ShapeDtypeStruct((B, featb.shape[1]), jnp.bfloat16),
            jax.ShapeDtypeStruct((B, PTC), jnp.float32),
        ),
        compiler_params=pltpu.CompilerParams(use_tc_tiling_on_sc=False),
        scratch_types=[
            pltpu.VMEM((CHUNK,), jnp.int32),
            pltpu.VMEM((CHUNK, featb.shape[1]), jnp.bfloat16),
            pltpu.VMEM((CHUNK, PTC), jnp.float32),
            pltpu.SemaphoreType.DMA,
            pltpu.SemaphoreType.DMA,
        ],
    )
    def gather_kernel(featb_hbm, coords_hbm, idx_hbm, outf_hbm, outp_hbm,
                      idx_v, rowsf_v, rowsp_v, sem_f, sem_p):
        wid = lax.axis_index("s") * NC + lax.axis_index("c")
        base = wid * per_w

        @pl.loop(0, n_chunks)
        def _(c):
            off = base + c * CHUNK
            pltpu.sync_copy(idx_hbm.at[pl.ds(off, CHUNK)], idx_v)
            cf = pltpu.async_copy(featb_hbm.at[idx_v], rowsf_v, sem_f)
            cp = pltpu.async_copy(coords_hbm.at[idx_v], rowsp_v, sem_p)
            cf.wait()
            cp.wait()
            pltpu.sync_copy(rowsf_v, outf_hbm.at[pl.ds(off, CHUNK)])
            pltpu.sync_copy(rowsp_v, outp_hbm.at[pl.ds(off, CHUNK)])

    return gather_kernel(featb, coords, idx)


def _make_tc_body(mb, d, k):
    def tc_body(gf_ref, gp_ref, outp_ref, kpt_ref, kv_ref, out_ref):
        featsb = gf_ref[...]
        pts = gp_ref[:, 0:3]
        op = outp_ref[...]
        opr = jnp.broadcast_to(op[:, None, :], (mb, d, 3)).reshape(mb * d, 3)
        sq = jnp.zeros((mb * d, 16), jnp.float32)
        for c in range(3):
            dc = pts[:, c:c + 1] - opr[:, c:c + 1]
            sq = sq + (dc - kpt_ref[c:c + 1, :]) ** 2
        w = jnp.maximum(1.0 - jnp.sqrt(sq) / EXTENT, 0.0)
        wb = w.astype(jnp.bfloat16)
        acc = jnp.zeros((mb, 128), jnp.float32)
        for j in range(k):
            p = wb[:, j:j + 1] * featsb
            wfj = p.reshape(mb, d, 128).sum(axis=1)
            acc = acc + jnp.dot(wfj.astype(jnp.float32), kv_ref[j],
                                preferred_element_type=jnp.float32)
        out_ref[...] = acc
    return tc_body


def kernel(points, features, output_points, neighbor_indices, k_points, k_values):
    n, f = features.shape
    m, d = neighbor_indices.shape
    k = k_values.shape[0]
    c_out = k_values.shape[2]

    featb = features.astype(jnp.bfloat16)
    coords = jnp.concatenate(
        [points, jnp.zeros((n, PTC - 3), jnp.float32)], axis=1)
    b = m * d
    grain = NW * CHUNK
    b_pad = ((b + grain - 1) // grain) * grain
    idx = jnp.pad(neighbor_indices.reshape(-1).astype(jnp.int32),
                  (0, b_pad - b))

    kpt = jnp.full((4, 16), 1e6, jnp.float32)
    kpt = kpt.at[0:3, 0:k].set(k_points.T)

    gf, gp = _sc_gather(featb, coords, idx)

    out = pl.pallas_call(
        _make_tc_body(MB, d, k),
        grid=(m // MB,),
        in_specs=[
            pl.BlockSpec((MB * d, f), lambda i: (i, 0)),
            pl.BlockSpec((MB * d, PTC), lambda i: (i, 0)),
            pl.BlockSpec((MB, 3), lambda i: (i, 0)),
            pl.BlockSpec((4, 16), lambda i: (0, 0)),
            pl.BlockSpec((k, f, c_out), lambda i: (0, 0, 0)),
        ],
        out_specs=pl.BlockSpec((MB, c_out), lambda i: (i, 0)),
        out_shape=jax.ShapeDtypeStruct((m, c_out), jnp.float32),
    )(gf, gp, output_points, kpt, k_values)
    return out
